# baseline (speedup 1.0000x reference)
import functools

import jax
import jax.numpy as jnp
from jax import lax
from jax.experimental import pallas as pl
from jax.experimental.pallas import tpu as pltpu
from jax.experimental.pallas import tpu_sc as plsc


@functools.lru_cache(maxsize=None)
def _build_gather(B, K, D):
    info = plsc.get_sparse_core_info()
    NC, NS = info.num_cores, info.num_subcores
    NW = NC * NS
    assert B % (8 * NW) == 0
    b_per_w = B // NW
    DP = 128
    n_chunks, nbuf = 8, 4
    assert b_per_w % n_chunks == 0
    chunk = b_per_w // n_chunks
    assert chunk % 8 == 0
    mesh = plsc.VectorSubcoreMesh(core_axis_name="c", subcore_axis_name="s")

    @functools.partial(
        pl.kernel,
        mesh=mesh,
        out_type=jax.ShapeDtypeStruct((B, DP), jnp.float32),
        scratch_types=[
            pltpu.VMEM((b_per_w,), jnp.int32),
            pltpu.VMEM((nbuf, chunk, DP), jnp.float32),
            pltpu.VMEM_SHARED((K, DP), jnp.float32),
            pltpu.SemaphoreType.DMA,
            pltpu.SemaphoreType.DMA,
            pltpu.SemaphoreType.DMA,
            pltpu.SemaphoreType.DMA,
        ],
    )
    def gather_kernel(y_hbm, table_hbm, out_hbm, idx_v, rows_v, table_sp,
                      isem, ssem, gsem, wsem):
        sid = lax.axis_index("s")
        wid = sid * NC + lax.axis_index("c")
        base = wid * b_per_w

        idx_cp = pltpu.make_async_copy(
            y_hbm.at[pl.ds(base, b_per_w)], idx_v, isem
        )
        idx_cp.start()

        r0 = 63 * sid - jnp.maximum(sid - 8, 0)
        n_rows = 63 - (sid >= 8).astype(jnp.int32)

        def _stage_row(i, _):
            pltpu.make_async_copy(
                table_hbm.at[r0 + i],
                table_sp.at[r0 + i, pl.ds(0, D)],
                ssem,
            ).start()
            return 0

        lax.fori_loop(0, n_rows, _stage_row, 0)

        def _drain_row(i, _):
            pltpu.make_async_copy(
                table_hbm.at[r0 + i],
                table_sp.at[r0 + i, pl.ds(0, D)],
                ssem,
            ).wait()
            return 0

        lax.fori_loop(0, n_rows, _drain_row, 0)

        plsc.subcore_barrier()
        idx_cp.wait()

        def _gather(c):
            return pltpu.make_async_copy(
                table_sp.at[idx_v.at[pl.ds(c * chunk, chunk)]],
                rows_v.at[c % nbuf],
                gsem,
            )

        def _write(c):
            return pltpu.make_async_copy(
                rows_v.at[c % nbuf],
                out_hbm.at[pl.ds(base + c * chunk, chunk)],
                wsem,
            )

        for c in range(nbuf):
            _gather(c).start()
        for c in range(n_chunks):
            _gather(c).wait()
            _write(c).start()
            if c + nbuf < n_chunks:
                _write(c).wait()
                _gather(c + nbuf).start()
        for c in range(n_chunks - nbuf, n_chunks):
            _write(c).wait()

    return gather_kernel


def kernel(y, codebook):
    (B,) = y.shape
    K, D = codebook.shape
    out = _build_gather(B, K, D)(y, codebook)
    return out[:, :D]


# speedup vs baseline: 1.2353x; 1.2353x over previous
import functools

import jax
import jax.numpy as jnp
from jax import lax
from jax.experimental import pallas as pl
from jax.experimental.pallas import tpu as pltpu
from jax.experimental.pallas import tpu_sc as plsc


@functools.lru_cache(maxsize=None)
def _build_gather(B, K, D):
    info = plsc.get_sparse_core_info()
    NC, NS = info.num_cores, info.num_subcores
    NW = NC * NS
    assert B % (8 * NW) == 0
    b_per_w = B // NW
    DP = 128
    n_chunks, nbuf = 8, 4
    assert b_per_w % n_chunks == 0
    chunk = b_per_w // n_chunks
    assert chunk % 8 == 0
    mesh = plsc.VectorSubcoreMesh(core_axis_name="c", subcore_axis_name="s")

    @functools.partial(
        pl.kernel,
        mesh=mesh,
        out_type=jax.ShapeDtypeStruct((B, DP), jnp.float32),
        scratch_types=[
            pltpu.VMEM((b_per_w,), jnp.int32),
            pltpu.VMEM((nbuf, chunk, DP), jnp.float32),
            pltpu.VMEM_SHARED((K, DP), jnp.float32),
            pltpu.SemaphoreType.DMA,
            pltpu.SemaphoreType.DMA,
            pltpu.SemaphoreType.DMA,
            pltpu.SemaphoreType.DMA,
        ],
    )
    def gather_kernel(y_hbm, table_hbm, out_hbm, idx_v, rows_v, table_sp,
                      isem, ssem, gsem, wsem):
        sid = lax.axis_index("s")
        wid = sid * NC + lax.axis_index("c")
        base = wid * b_per_w

        idx_cp = pltpu.make_async_copy(
            y_hbm.at[pl.ds(base, b_per_w)], idx_v, isem
        )
        idx_cp.start()

        @pl.when(sid < 15)
        def _stage_lo():
            pltpu.sync_copy(
                table_hbm.at[pl.ds(sid * 64, 64)],
                table_sp.at[pl.ds(sid * 64, 64)],
            )

        @pl.when(sid == 15)
        def _stage_hi():
            pltpu.sync_copy(
                table_hbm.at[pl.ds(960, K - 960)],
                table_sp.at[pl.ds(960, K - 960)],
            )

        plsc.subcore_barrier()
        idx_cp.wait()

        def _gather(c):
            return pltpu.make_async_copy(
                table_sp.at[idx_v.at[pl.ds(c * chunk, chunk)]],
                rows_v.at[c % nbuf],
                gsem,
            )

        def _write(c):
            return pltpu.make_async_copy(
                rows_v.at[c % nbuf],
                out_hbm.at[pl.ds(base + c * chunk, chunk)],
                wsem,
            )

        for c in range(nbuf):
            _gather(c).start()
        for c in range(n_chunks):
            _gather(c).wait()
            _write(c).start()
            if c + nbuf < n_chunks:
                _write(c).wait()
                _gather(c + nbuf).start()
        for c in range(n_chunks - nbuf, n_chunks):
            _write(c).wait()

    return gather_kernel


def kernel(y, codebook):
    (B,) = y.shape
    K, D = codebook.shape
    DP = 128
    table = jnp.concatenate(
        [codebook, jnp.zeros((K, DP - D), jnp.float32)], axis=1
    )
    out = _build_gather(B, K, D)(y, table)
    return out[:, :D]
